# baseline (device time: 128428 ns/iter reference)
import jax
import jax.numpy as jnp
from jax import lax
from jax.experimental import pallas as pl
from jax.experimental.pallas import tpu as pltpu

N_DEV = 4
B, SQ, SKV_L, HQ, DH = 2, 512, 512, 32, 64
H_L = HQ // N_DEV
HD_L = H_L * DH
SKV = SKV_L * N_DEV
D_MODEL = 768
BLK = 64
C = SQ // N_DEV

_DIT = getattr(pl, "DeviceIdType", None) or pltpu.DeviceIdType
_sem_signal = getattr(pl, "semaphore_signal", None) or pltpu.semaphore_signal
_sem_wait = getattr(pl, "semaphore_wait", None) or pltpu.semaphore_wait
_CP = getattr(pltpu, "CompilerParams", None) or pltpu.TPUCompilerParams

BF = jnp.bfloat16


def kernel(x, Wq, K_ext, V_ext, Wo):
    def body(x_ref, wq_ref, k_ref, v_ref, wo_ref, out_ref,
             q_buf, kg, vg, ctx_buf, acc_buf, lbuf, bias4, psend, ar1_buf,
             pbuf, kr_sems, vr_sems, ks_sems, vs_sems,
             ar1_s, ar1_r, ar2_s, ar2_r):
        me = lax.axis_index("i")

        barrier_sem = pltpu.get_barrier_semaphore()
        for d in range(1, N_DEV):
            peer = lax.rem(me + d, N_DEV)
            _sem_signal(barrier_sem, inc=1, device_id=(peer,),
                        device_id_type=_DIT.MESH)
        _sem_wait(barrier_sem, N_DEV - 1)

        sends = []
        for d in range(1, N_DEV):
            j = lax.rem(me + d, N_DEV)
            for src_ref, dst_g, ssem, rsem in (
                (k_ref, kg, ks_sems, kr_sems),
                (v_ref, vg, vs_sems, vr_sems),
            ):
                rdma = pltpu.make_async_remote_copy(
                    src_ref=src_ref.at[:, :, pl.ds(j * HD_L, HD_L)],
                    dst_ref=dst_g.at[:, pl.ds(me * SKV_L, SKV_L), :],
                    send_sem=ssem.at[j],
                    recv_sem=rsem.at[me],
                    device_id=(j,),
                    device_id_type=_DIT.MESH,
                )
                rdma.start()
                sends.append(rdma)

        kg[:, pl.ds(me * SKV_L, SKV_L), :] = k_ref[:, :, pl.ds(me * HD_L, HD_L)]
        vg[:, pl.ds(me * SKV_L, SKV_L), :] = v_ref[:, :, pl.ds(me * HD_L, HD_L)]

        for b in range(B):
            q_buf[b] = (jnp.dot(x_ref[b], wq_ref[...],
                                preferred_element_type=jnp.float32)
                        * 0.125).astype(BF)

        for c in range(N_DEV):
            qb = lax.broadcasted_iota(jnp.int32, (SQ, SKV_L), 0) // BLK
            kb = lax.broadcasted_iota(jnp.int32, (SQ, SKV_L), 1) // BLK + 8 * c
            mask = (qb == kb) | (kb == 0) | (lax.rem(qb + kb, 3) == 0)
            bias4[c] = jnp.where(mask, 0.0, -1e9).astype(BF)

        ones_col = jnp.ones((SKV_L, 8), BF)

        for idx, d in enumerate((0, 1, 3, 2)):
            j = lax.rem(me + d, N_DEV)
            if d != 0:
                for dst_g, ssem, rsem in ((kg, ks_sems, kr_sems),
                                          (vg, vs_sems, vr_sems)):
                    recv = pltpu.make_async_remote_copy(
                        src_ref=dst_g.at[:, pl.ds(j * SKV_L, SKV_L), :],
                        dst_ref=dst_g.at[:, pl.ds(j * SKV_L, SKV_L), :],
                        send_sem=ssem.at[j],
                        recv_sem=rsem.at[j],
                        device_id=(j,),
                        device_id_type=_DIT.MESH,
                    )
                    recv.wait_recv()
            for b in range(B):
                for h in range(H_L):
                    k = kg[b, pl.ds(j * SKV_L, SKV_L),
                           h * DH:(h + 1) * DH]
                    v = jnp.concatenate(
                        [vg[b, pl.ds(j * SKV_L, SKV_L),
                            h * DH:(h + 1) * DH], ones_col], axis=1)
                    for t in range(SQ // C):
                        ts = slice(t * C, (t + 1) * C)
                        q = q_buf[b, ts, h * DH:(h + 1) * DH]
                        s = lax.dot_general(
                            q, k, (((1,), (1,)), ((), ())),
                            preferred_element_type=jnp.float32,
                        ).astype(BF) + bias4[j, ts, :]
                        w = jnp.exp(s)
                        awl = jnp.dot(w, v,
                                      preferred_element_type=jnp.float32)
                        if idx == 0:
                            acc_buf[b, ts, h * DH:(h + 1) * DH] = awl[:, :DH]
                            lbuf[b, h, ts] = awl[:, DH:DH + 1]
                        else:
                            acc_buf[b, ts, h * DH:(h + 1) * DH] += awl[:, :DH]
                            lbuf[b, h, ts] += awl[:, DH:DH + 1]

        for b in range(B):
            for h in range(H_L):
                ctx_buf[b, :, h * DH:(h + 1) * DH] = (
                    acc_buf[b, :, h * DH:(h + 1) * DH] / lbuf[b, h]
                ).astype(BF)

        for d in range(1, N_DEV):
            t = lax.rem(me + d, N_DEV)
            for b in range(B):
                psend[d - 1, b] = jnp.dot(
                    ctx_buf[b, pl.ds(t * C, C), :], wo_ref[...],
                    preferred_element_type=jnp.float32).astype(BF)
            rdma = pltpu.make_async_remote_copy(
                src_ref=psend.at[d - 1],
                dst_ref=ar1_buf.at[d - 1],
                send_sem=ar1_s.at[d - 1],
                recv_sem=ar1_r.at[d - 1],
                device_id=(t,),
                device_id_type=_DIT.MESH,
            )
            rdma.start()
            sends.append(rdma)
        for b in range(B):
            psend[N_DEV - 1, b] = jnp.dot(
                ctx_buf[b, pl.ds(me * C, C), :], wo_ref[...],
                preferred_element_type=jnp.float32).astype(BF)

        for d in range(1, N_DEV):
            j = lax.rem(me - d + N_DEV, N_DEV)
            recv = pltpu.make_async_remote_copy(
                src_ref=ar1_buf.at[d - 1], dst_ref=ar1_buf.at[d - 1],
                send_sem=ar1_s.at[d - 1], recv_sem=ar1_r.at[d - 1],
                device_id=(j,), device_id_type=_DIT.MESH,
            )
            recv.wait_recv()
        red = (psend[N_DEV - 1].astype(jnp.float32)
               + ar1_buf[0].astype(jnp.float32)
               + ar1_buf[1].astype(jnp.float32)
               + ar1_buf[2].astype(jnp.float32)).astype(BF)
        pbuf[:, pl.ds(me * C, C), :] = red

        for d in range(1, N_DEV):
            t = lax.rem(me + d, N_DEV)
            rdma = pltpu.make_async_remote_copy(
                src_ref=pbuf.at[:, pl.ds(me * C, C), :],
                dst_ref=pbuf.at[:, pl.ds(me * C, C), :],
                send_sem=ar2_s.at[d - 1],
                recv_sem=ar2_r.at[d - 1],
                device_id=(t,),
                device_id_type=_DIT.MESH,
            )
            rdma.start()
            sends.append(rdma)
        for d in range(1, N_DEV):
            j = lax.rem(me - d + N_DEV, N_DEV)
            recv = pltpu.make_async_remote_copy(
                src_ref=pbuf.at[:, pl.ds(j * C, C), :],
                dst_ref=pbuf.at[:, pl.ds(j * C, C), :],
                send_sem=ar2_s.at[d - 1], recv_sem=ar2_r.at[d - 1],
                device_id=(j,), device_id_type=_DIT.MESH,
            )
            recv.wait_recv()

        out_ref[...] = pbuf[...].astype(jnp.float32)

        for rdma in sends:
            rdma.wait_send()

    kv_shape = (B, SKV_L, HQ * DH)
    return pl.pallas_call(
        body,
        out_shape=jax.ShapeDtypeStruct((B, SQ, D_MODEL), jnp.float32),
        in_specs=[pl.BlockSpec(memory_space=pltpu.VMEM)] * 5,
        out_specs=pl.BlockSpec(memory_space=pltpu.VMEM),
        scratch_shapes=[
            pltpu.VMEM((B, SQ, HD_L), BF),
            pltpu.VMEM((B, SKV, HD_L), BF),
            pltpu.VMEM((B, SKV, HD_L), BF),
            pltpu.VMEM((B, SQ, HD_L), BF),
            pltpu.VMEM((B, SQ, HD_L), jnp.float32),
            pltpu.VMEM((B, H_L, SQ, 1), jnp.float32),
            pltpu.VMEM((N_DEV, SQ, SKV_L), BF),
            pltpu.VMEM((N_DEV, B, C, D_MODEL), BF),
            pltpu.VMEM((N_DEV - 1, B, C, D_MODEL), BF),
            pltpu.VMEM((B, SQ, D_MODEL), BF),
            pltpu.SemaphoreType.DMA((N_DEV,)),
            pltpu.SemaphoreType.DMA((N_DEV,)),
            pltpu.SemaphoreType.DMA((N_DEV,)),
            pltpu.SemaphoreType.DMA((N_DEV,)),
            pltpu.SemaphoreType.DMA((N_DEV - 1,)),
            pltpu.SemaphoreType.DMA((N_DEV - 1,)),
            pltpu.SemaphoreType.DMA((N_DEV - 1,)),
            pltpu.SemaphoreType.DMA((N_DEV - 1,)),
        ],
        compiler_params=_CP(collective_id=0),
    )(x.astype(BF), Wq.astype(BF),
      K_ext.reshape(kv_shape).astype(BF),
      V_ext.reshape(kv_shape).astype(BF),
      Wo.astype(BF))


# device time: 75492 ns/iter; 1.7012x vs baseline; 1.7012x over previous
import os

import jax
import jax.numpy as jnp
from jax import lax
from jax.experimental import pallas as pl
from jax.experimental.pallas import tpu as pltpu

N_DEV = 4
B, SQ, SKV_L, HQ, DH = 2, 512, 512, 32, 64
H_L = HQ // N_DEV
HD_L = H_L * DH
SKV = SKV_L * N_DEV
D_MODEL = 768
BLK = 64
C = SQ // N_DEV

_DIT = getattr(pl, "DeviceIdType", None) or pltpu.DeviceIdType
_sem_signal = getattr(pl, "semaphore_signal", None) or pltpu.semaphore_signal
_sem_wait = getattr(pl, "semaphore_wait", None) or pltpu.semaphore_wait
_CP = getattr(pltpu, "CompilerParams", None) or pltpu.TPUCompilerParams

BF = jnp.bfloat16

_VARIANT = os.environ.get("KVAR", "full")


def kernel(x, Wq, K_ext, V_ext, Wo):
    def body(x_ref, wq_ref, k_ref, v_ref, wo_ref, out_ref,
             q_buf, kg, vg, ctx_buf, acc_buf, lbuf, bias4, psend, ar1_buf,
             pbuf, kr_sems, vr_sems, ks_sems, vs_sems,
             ar1_s, ar1_r, ar2_s, ar2_r):
        me = lax.axis_index("i")

        barrier_sem = pltpu.get_barrier_semaphore()
        for d in range(1, N_DEV):
            peer = lax.rem(me + d, N_DEV)
            _sem_signal(barrier_sem, inc=1, device_id=(peer,),
                        device_id_type=_DIT.MESH)
        _sem_wait(barrier_sem, N_DEV - 1)

        sends = []
        for d in range(1, N_DEV) if _VARIANT != "nocomm" else ():
            j = lax.rem(me + d, N_DEV)
            for src_ref, dst_g, ssem, rsem in (
                (k_ref, kg, ks_sems, kr_sems),
                (v_ref, vg, vs_sems, vr_sems),
            ):
                rdma = pltpu.make_async_remote_copy(
                    src_ref=src_ref.at[:, :, pl.ds(j * HD_L, HD_L)],
                    dst_ref=dst_g.at[:, pl.ds(me * SKV_L, SKV_L), :],
                    send_sem=ssem.at[j],
                    recv_sem=rsem.at[me],
                    device_id=(j,),
                    device_id_type=_DIT.MESH,
                )
                rdma.start()
                sends.append(rdma)

        kg[:, pl.ds(me * SKV_L, SKV_L), :] = k_ref[:, :, pl.ds(me * HD_L, HD_L)]
        vg[:, pl.ds(me * SKV_L, SKV_L), :] = v_ref[:, :, pl.ds(me * HD_L, HD_L)]

        for b in range(B):
            q_buf[b] = (jnp.dot(x_ref[b], wq_ref[...],
                                preferred_element_type=jnp.float32)
                        * 0.125).astype(BF)

        for c in range(N_DEV):
            qb = lax.broadcasted_iota(jnp.int32, (SQ, SKV_L), 0) // BLK
            kb = lax.broadcasted_iota(jnp.int32, (SQ, SKV_L), 1) // BLK + 8 * c
            mask = (qb == kb) | (kb == 0) | (lax.rem(qb + kb, 3) == 0)
            bias4[c] = jnp.where(mask, 0.0, -1e9).astype(BF)

        ones_col = jnp.ones((SKV_L, 8), BF)

        for idx, d in enumerate((0, 1, 3, 2)):
            j = lax.rem(me + d, N_DEV) if _VARIANT != "nocomm" else me
            if d != 0 and _VARIANT != "nocomm":
                for dst_g, ssem, rsem in ((kg, ks_sems, kr_sems),
                                          (vg, vs_sems, vr_sems)):
                    recv = pltpu.make_async_remote_copy(
                        src_ref=dst_g.at[:, pl.ds(j * SKV_L, SKV_L), :],
                        dst_ref=dst_g.at[:, pl.ds(j * SKV_L, SKV_L), :],
                        send_sem=ssem.at[j],
                        recv_sem=rsem.at[j],
                        device_id=(j,),
                        device_id_type=_DIT.MESH,
                    )
                    recv.wait_recv()
            for b in range(B):
                for h in range(H_L):
                    k = kg[b, pl.ds(j * SKV_L, SKV_L),
                           h * DH:(h + 1) * DH]
                    v = jnp.concatenate(
                        [vg[b, pl.ds(j * SKV_L, SKV_L),
                            h * DH:(h + 1) * DH], ones_col], axis=1)
                    for t in range(SQ // C):
                        ts = slice(t * C, (t + 1) * C)
                        q = q_buf[b, ts, h * DH:(h + 1) * DH]
                        s = lax.dot_general(
                            q, k, (((1,), (1,)), ((), ())),
                            preferred_element_type=jnp.float32,
                        ).astype(BF) + bias4[j, ts, :]
                        w = jnp.exp(s)
                        awl = jnp.dot(w, v,
                                      preferred_element_type=jnp.float32)
                        if idx == 0:
                            acc_buf[b, ts, h * DH:(h + 1) * DH] = awl[:, :DH]
                            lbuf[b, h, ts] = awl[:, DH:DH + 1]
                        else:
                            acc_buf[b, ts, h * DH:(h + 1) * DH] += awl[:, :DH]
                            lbuf[b, h, ts] += awl[:, DH:DH + 1]

        for b in range(B):
            for h in range(H_L):
                ctx_buf[b, :, h * DH:(h + 1) * DH] = (
                    acc_buf[b, :, h * DH:(h + 1) * DH] / lbuf[b, h]
                ).astype(BF)

        if _VARIANT != "full":
            for b in range(B):
                out_ref[b] = jnp.dot(ctx_buf[b], wo_ref[...],
                                     preferred_element_type=jnp.float32)
            for rdma in sends:
                rdma.wait_send()
            return

        for d in range(1, N_DEV):
            t = lax.rem(me + d, N_DEV)
            for b in range(B):
                psend[d - 1, b] = jnp.dot(
                    ctx_buf[b, pl.ds(t * C, C), :], wo_ref[...],
                    preferred_element_type=jnp.float32).astype(BF)
            rdma = pltpu.make_async_remote_copy(
                src_ref=psend.at[d - 1],
                dst_ref=ar1_buf.at[d - 1],
                send_sem=ar1_s.at[d - 1],
                recv_sem=ar1_r.at[d - 1],
                device_id=(t,),
                device_id_type=_DIT.MESH,
            )
            rdma.start()
            sends.append(rdma)
        for b in range(B):
            psend[N_DEV - 1, b] = jnp.dot(
                ctx_buf[b, pl.ds(me * C, C), :], wo_ref[...],
                preferred_element_type=jnp.float32).astype(BF)

        for d in range(1, N_DEV):
            j = lax.rem(me - d + N_DEV, N_DEV)
            recv = pltpu.make_async_remote_copy(
                src_ref=ar1_buf.at[d - 1], dst_ref=ar1_buf.at[d - 1],
                send_sem=ar1_s.at[d - 1], recv_sem=ar1_r.at[d - 1],
                device_id=(j,), device_id_type=_DIT.MESH,
            )
            recv.wait_recv()
        red = (psend[N_DEV - 1].astype(jnp.float32)
               + ar1_buf[0].astype(jnp.float32)
               + ar1_buf[1].astype(jnp.float32)
               + ar1_buf[2].astype(jnp.float32)).astype(BF)
        pbuf[:, pl.ds(me * C, C), :] = red

        for d in range(1, N_DEV):
            t = lax.rem(me + d, N_DEV)
            rdma = pltpu.make_async_remote_copy(
                src_ref=pbuf.at[:, pl.ds(me * C, C), :],
                dst_ref=pbuf.at[:, pl.ds(me * C, C), :],
                send_sem=ar2_s.at[d - 1],
                recv_sem=ar2_r.at[d - 1],
                device_id=(t,),
                device_id_type=_DIT.MESH,
            )
            rdma.start()
            sends.append(rdma)
        for d in range(1, N_DEV):
            j = lax.rem(me - d + N_DEV, N_DEV)
            recv = pltpu.make_async_remote_copy(
                src_ref=pbuf.at[:, pl.ds(j * C, C), :],
                dst_ref=pbuf.at[:, pl.ds(j * C, C), :],
                send_sem=ar2_s.at[d - 1], recv_sem=ar2_r.at[d - 1],
                device_id=(j,), device_id_type=_DIT.MESH,
            )
            recv.wait_recv()

        out_ref[...] = pbuf[...].astype(jnp.float32)

        for rdma in sends:
            rdma.wait_send()

    kv_shape = (B, SKV_L, HQ * DH)
    return pl.pallas_call(
        body,
        out_shape=jax.ShapeDtypeStruct((B, SQ, D_MODEL), jnp.float32),
        in_specs=[pl.BlockSpec(memory_space=pltpu.VMEM)] * 5,
        out_specs=pl.BlockSpec(memory_space=pltpu.VMEM),
        scratch_shapes=[
            pltpu.VMEM((B, SQ, HD_L), BF),
            pltpu.VMEM((B, SKV, HD_L), BF),
            pltpu.VMEM((B, SKV, HD_L), BF),
            pltpu.VMEM((B, SQ, HD_L), BF),
            pltpu.VMEM((B, SQ, HD_L), jnp.float32),
            pltpu.VMEM((B, H_L, SQ, 1), jnp.float32),
            pltpu.VMEM((N_DEV, SQ, SKV_L), BF),
            pltpu.VMEM((N_DEV, B, C, D_MODEL), BF),
            pltpu.VMEM((N_DEV - 1, B, C, D_MODEL), BF),
            pltpu.VMEM((B, SQ, D_MODEL), BF),
            pltpu.SemaphoreType.DMA((N_DEV,)),
            pltpu.SemaphoreType.DMA((N_DEV,)),
            pltpu.SemaphoreType.DMA((N_DEV,)),
            pltpu.SemaphoreType.DMA((N_DEV,)),
            pltpu.SemaphoreType.DMA((N_DEV - 1,)),
            pltpu.SemaphoreType.DMA((N_DEV - 1,)),
            pltpu.SemaphoreType.DMA((N_DEV - 1,)),
            pltpu.SemaphoreType.DMA((N_DEV - 1,)),
        ],
        compiler_params=_CP(collective_id=0),
    )(x.astype(BF), Wq.astype(BF),
      K_ext.reshape(kv_shape).astype(BF),
      V_ext.reshape(kv_shape).astype(BF),
      Wo.astype(BF))
